# baseline (speedup 1.0000x reference)
import functools

import jax
import jax.numpy as jnp
from jax import lax
from jax.experimental import pallas as pl
from jax.experimental.pallas import tpu as pltpu
from jax.experimental.pallas import tpu_sc as plsc

VOCAB = 1000
VPAD = 1024
B, L = 1024, 20
TOK = B * L
NVT = VOCAB // 8
NBT = B // 128
OUTW = L * NVT * NBT * 8 * 128

NC, NS = 2, 16
NW = NC * NS
VPW = VPAD // NW
BPW = TOK // NW
LANES = 16

BLK = 256
NBLK = TOK // BLK


CHW = NBT * 1024


def _sc_gather_body(tt_hbm, tflat_hbm, xt_hbm, x_hbm, y_hbm,
                    out_hbm, part_hbm,
                    tbl_v, xt_v, x_v, y_v, pbuf0, pbuf1, pidx_v, pval_v,
                    pick_v, wsem0, wsem1, psem):
    wid = lax.axis_index("s") * NC + lax.axis_index("c")

    tbase = pl.multiple_of(wid * (VPW * VPAD), VPW * VPAD)
    pltpu.sync_copy(tt_hbm.at[pl.ds(tbase, VPW * VPAD)], tbl_v)
    pltpu.sync_copy(xt_hbm, xt_v)

    def drain(buf, sem):
        pltpu.make_async_copy(buf, out_hbm.at[pl.ds(0, CHW)], sem).wait()

    def fill_and_fire(ci, l, buf, sem):
        @plsc.parallel_loop(0, NBT, unroll=2)
        def bt_body(bt):
            xoff = l * B + bt * 128
            poff = bt * 1024
            for h in range(2):
                xs = [xt_v[pl.ds(xoff + (h * 4 + q4) * LANES, LANES)]
                      for q4 in range(4)]
                gs = []
                for q4 in range(4):
                    for vs in range(8):
                        g16 = plsc.load_gather(
                            tbl_v,
                            [xs[q4] + jnp.int32((ci * 8 + vs) * 1024)])
                        gs.append((h * 4 + q4, vs, g16))
                for q, vs, g16 in gs:
                    buf[pl.ds(poff + vs * 128 + q * LANES, LANES)] = g16
        vti = wid * (VPW // 8) + ci
        woff = (l * NVT + vti) * CHW
        return pltpu.async_copy(buf, out_hbm.at[pl.ds(woff, CHW)], sem)

    for ci in range(VPW // 8):
        vti = wid * (VPW // 8) + ci

        @pl.when(vti < NVT)
        def _chunks():
            def lp_body(k, carry):
                @pl.when((k > 0) | jnp.bool_(ci > 0))
                def _w0():
                    drain(pbuf0, wsem0)
                fill_and_fire(ci, 2 * k, pbuf0, wsem0)

                @pl.when((k > 0) | jnp.bool_(ci > 0))
                def _w1():
                    drain(pbuf1, wsem1)
                fill_and_fire(ci, 2 * k + 1, pbuf1, wsem1)
                return carry

            lax.fori_loop(0, L // 2, lp_body, 0)

    drain(pbuf0, wsem0)
    drain(pbuf1, wsem1)

    base = pl.multiple_of(wid * BPW, BPW)
    pltpu.sync_copy(x_hbm.at[pl.ds(base, BPW)], x_v)
    pltpu.sync_copy(y_hbm.at[pl.ds(base, BPW)], y_v)
    for q in range(BPW // LANES):
        x16 = x_v[pl.ds(q * LANES, LANES)]
        y16 = y_v[pl.ds(q * LANES, LANES)]
        pidx_v[pl.ds(q * LANES, LANES)] = x16 * VOCAB + y16
    cps = [pltpu.async_copy(tflat_hbm.at[pidx_v.at[pl.ds(c * 128, 128)]],
                            pval_v.at[pl.ds(c * 128, 128)], psem)
           for c in range(BPW // 128)]
    acc = jnp.zeros((LANES,), jnp.float32)
    for cp in cps:
        cp.wait()
    for q in range(BPW // LANES):
        acc = acc + pval_v[pl.ds(q * LANES, LANES)]

    pick_v[...] = acc
    pltpu.sync_copy(pick_v, part_hbm.at[wid])


_sc_gather = functools.partial(
    pl.kernel,
    out_type=[
        jax.ShapeDtypeStruct((OUTW,), jnp.float32),
        jax.ShapeDtypeStruct((NW, LANES), jnp.float32),
    ],
    mesh=plsc.VectorSubcoreMesh(
        core_axis_name="c", subcore_axis_name="s",
        num_cores=NC, num_subcores=NS),
    compiler_params=pltpu.CompilerParams(
        use_tc_tiling_on_sc=False, needs_layout_passes=False),
    scratch_types=[
        pltpu.VMEM((VPW * VPAD,), jnp.float32),
        pltpu.VMEM((TOK,), jnp.int32),
        pltpu.VMEM((BPW,), jnp.int32),
        pltpu.VMEM((BPW,), jnp.int32),
        pltpu.VMEM((CHW,), jnp.float32),
        pltpu.VMEM((CHW,), jnp.float32),
        pltpu.VMEM((BPW,), jnp.int32),
        pltpu.VMEM((BPW,), jnp.float32),
        pltpu.VMEM((LANES,), jnp.float32),
        pltpu.SemaphoreType.DMA,
        pltpu.SemaphoreType.DMA,
        pltpu.SemaphoreType.DMA,
    ],
)(_sc_gather_body)


def _lse_body(x_ref, table_ref, out_ref, lse_scr, acc_scr):
    pid = pl.program_id(0)
    t = table_ref[...]

    @pl.when(pid == 0)
    def _init():
        m = jnp.max(t, axis=1, keepdims=True)
        s = jnp.sum(jnp.exp(t - m), axis=1, keepdims=True)
        lse_scr[...] = m + jnp.log(s)
        acc_scr[0] = 0.0

    xv = x_ref[0]
    iota = lax.broadcasted_iota(jnp.int32, (BLK, VOCAB), 1)
    oh_x = (xv == iota).astype(jnp.float32)
    lse_tok = lax.dot_general(
        oh_x, lse_scr[...], (((1,), (0,)), ((), ())),
        preferred_element_type=jnp.float32)
    acc_scr[0] += jnp.sum(lse_tok)

    @pl.when(pid == NBLK - 1)
    def _fin():
        out_ref[...] = jnp.full((1, 1), acc_scr[0], jnp.float32)


def _lse_sum(Xr, table):
    return pl.pallas_call(
        _lse_body,
        grid=(NBLK,),
        in_specs=[
            pl.BlockSpec((1, BLK, 1), lambda i: (i, 0, 0)),
            pl.BlockSpec((VOCAB, VOCAB), lambda i: (0, 0)),
        ],
        out_specs=pl.BlockSpec((1, 1), lambda i: (0, 0)),
        out_shape=jax.ShapeDtypeStruct((1, 1), jnp.float32),
        scratch_shapes=[
            pltpu.VMEM((VOCAB, 1), jnp.float32),
            pltpu.SMEM((1,), jnp.float32),
        ],
    )(Xr, table)


def kernel(X, Y, table):
    Xi = X.astype(jnp.int32)
    Xf = Xi.reshape(TOK)
    Yf = Y.astype(jnp.int32).reshape(TOK)
    XT = Xi.T.reshape(TOK)
    tt8 = (jnp.pad(table.T, ((0, VPAD - VOCAB), (0, VPAD - VOCAB)))
           .reshape(VPAD * VPAD))

    out1, parts = _sc_gather(tt8, table.reshape(VOCAB * VOCAB),
                             XT, Xf, Yf)
    lse_sum = _lse_sum(Xi.reshape(NBLK, BLK, 1), table)
    loss = (lse_sum[0, 0] - jnp.sum(parts)) / TOK

    logits = (out1.reshape(L, NVT, NBT, 8, 128)
              .transpose(2, 4, 0, 1, 3)
              .reshape(B, L, VOCAB))
    return logits, loss


# speedup vs baseline: 1.1649x; 1.1649x over previous
import functools

import jax
import jax.numpy as jnp
from jax import lax
from jax.experimental import pallas as pl
from jax.experimental.pallas import tpu as pltpu
from jax.experimental.pallas import tpu_sc as plsc

VOCAB = 1000
VPAD = 1024
B, L = 1024, 20
TOK = B * L
NVT = VOCAB // 8
NBT = B // 128
OUTW = L * NVT * NBT * 8 * 128

NC, NS = 2, 16
NW = NC * NS
VPW = VPAD // NW
BPW = TOK // NW
LANES = 16

BLK = 256
NBLK = TOK // BLK


CHW = NBT * 1024


def _sc_gather_body(tt_hbm, tflat_hbm, xt_hbm, x_hbm, y_hbm,
                    out_hbm, part_hbm,
                    tbl_v, xt_v, x_v, y_v, pbuf0, pbuf1, pidx_v, pval_v,
                    pick_v, wsem0, wsem1, psem):
    wid = lax.axis_index("s") * NC + lax.axis_index("c")

    tbase = pl.multiple_of(wid * (VPW * VPAD), VPW * VPAD)
    pltpu.sync_copy(tt_hbm.at[pl.ds(tbase, VPW * VPAD)], tbl_v)
    pltpu.sync_copy(xt_hbm, xt_v)

    def drain(buf, sem):
        pltpu.make_async_copy(buf, out_hbm.at[pl.ds(0, CHW)], sem).wait()

    def fill_and_fire(ci, l, buf, sem):
        @plsc.parallel_loop(0, NBT)
        def bt_body(bt):
            xoff = l * B + bt * 128
            poff = bt * 1024
            for h in range(2):
                xs = [xt_v[pl.ds(xoff + (h * 4 + q4) * LANES, LANES)]
                      for q4 in range(4)]
                gs = []
                for q4 in range(4):
                    for vs in range(8):
                        g16 = plsc.load_gather(
                            tbl_v,
                            [xs[q4] + jnp.int32((ci * 8 + vs) * 1024)])
                        gs.append((h * 4 + q4, vs, g16))
                for q, vs, g16 in gs:
                    buf[pl.ds(poff + vs * 128 + q * LANES, LANES)] = g16
        vti = wid * (VPW // 8) + ci
        woff = (l * NVT + vti) * CHW
        return pltpu.async_copy(buf, out_hbm.at[pl.ds(woff, CHW)], sem)

    for ci in range(VPW // 8):
        vti = wid * (VPW // 8) + ci

        @pl.when(vti < NVT)
        def _chunks():
            def lp_body(k, carry):
                @pl.when((k > 0) | jnp.bool_(ci > 0))
                def _w0():
                    drain(pbuf0, wsem0)
                fill_and_fire(ci, 2 * k, pbuf0, wsem0)

                @pl.when((k > 0) | jnp.bool_(ci > 0))
                def _w1():
                    drain(pbuf1, wsem1)
                fill_and_fire(ci, 2 * k + 1, pbuf1, wsem1)
                return carry

            lax.fori_loop(0, L // 2, lp_body, 0)

    drain(pbuf0, wsem0)
    drain(pbuf1, wsem1)

    base = pl.multiple_of(wid * BPW, BPW)
    pltpu.sync_copy(x_hbm.at[pl.ds(base, BPW)], x_v)
    pltpu.sync_copy(y_hbm.at[pl.ds(base, BPW)], y_v)
    for q in range(BPW // LANES):
        x16 = x_v[pl.ds(q * LANES, LANES)]
        y16 = y_v[pl.ds(q * LANES, LANES)]
        pidx_v[pl.ds(q * LANES, LANES)] = x16 * VOCAB + y16
    cps = [pltpu.async_copy(tflat_hbm.at[pidx_v.at[pl.ds(c * 128, 128)]],
                            pval_v.at[pl.ds(c * 128, 128)], psem)
           for c in range(BPW // 128)]
    acc = jnp.zeros((LANES,), jnp.float32)
    for cp in cps:
        cp.wait()
    for q in range(BPW // LANES):
        acc = acc + pval_v[pl.ds(q * LANES, LANES)]

    pick_v[...] = acc
    pltpu.sync_copy(pick_v, part_hbm.at[wid])


_sc_gather = functools.partial(
    pl.kernel,
    out_type=[
        jax.ShapeDtypeStruct((OUTW,), jnp.float32),
        jax.ShapeDtypeStruct((NW, LANES), jnp.float32),
    ],
    mesh=plsc.VectorSubcoreMesh(
        core_axis_name="c", subcore_axis_name="s",
        num_cores=NC, num_subcores=NS),
    compiler_params=pltpu.CompilerParams(
        use_tc_tiling_on_sc=False, needs_layout_passes=False),
    scratch_types=[
        pltpu.VMEM((VPW * VPAD,), jnp.float32),
        pltpu.VMEM((TOK,), jnp.int32),
        pltpu.VMEM((BPW,), jnp.int32),
        pltpu.VMEM((BPW,), jnp.int32),
        pltpu.VMEM((CHW,), jnp.float32),
        pltpu.VMEM((CHW,), jnp.float32),
        pltpu.VMEM((BPW,), jnp.int32),
        pltpu.VMEM((BPW,), jnp.float32),
        pltpu.VMEM((LANES,), jnp.float32),
        pltpu.SemaphoreType.DMA,
        pltpu.SemaphoreType.DMA,
        pltpu.SemaphoreType.DMA,
    ],
)(_sc_gather_body)


def _lse_body(x_ref, table_ref, out_ref, lse_scr, acc_scr):
    pid = pl.program_id(0)
    t = table_ref[...]

    @pl.when(pid == 0)
    def _init():
        m = jnp.max(t, axis=1, keepdims=True)
        s = jnp.sum(jnp.exp(t - m), axis=1, keepdims=True)
        lse_scr[...] = m + jnp.log(s)
        acc_scr[0] = 0.0

    xv = x_ref[0]
    iota = lax.broadcasted_iota(jnp.int32, (BLK, VOCAB), 1)
    oh_x = (xv == iota).astype(jnp.float32)
    lse_tok = lax.dot_general(
        oh_x, lse_scr[...], (((1,), (0,)), ((), ())),
        preferred_element_type=jnp.float32)
    acc_scr[0] += jnp.sum(lse_tok)

    @pl.when(pid == NBLK - 1)
    def _fin():
        out_ref[...] = jnp.full((1, 1), acc_scr[0], jnp.float32)


def _lse_sum(Xr, table):
    return pl.pallas_call(
        _lse_body,
        grid=(NBLK,),
        in_specs=[
            pl.BlockSpec((1, BLK, 1), lambda i: (i, 0, 0)),
            pl.BlockSpec((VOCAB, VOCAB), lambda i: (0, 0)),
        ],
        out_specs=pl.BlockSpec((1, 1), lambda i: (0, 0)),
        out_shape=jax.ShapeDtypeStruct((1, 1), jnp.float32),
        scratch_shapes=[
            pltpu.VMEM((VOCAB, 1), jnp.float32),
            pltpu.SMEM((1,), jnp.float32),
        ],
    )(Xr, table)


def kernel(X, Y, table):
    Xi = X.astype(jnp.int32)
    Xf = Xi.reshape(TOK)
    Yf = Y.astype(jnp.int32).reshape(TOK)
    XT = Xi.T.reshape(TOK)
    tt8 = (jnp.pad(table.T, ((0, VPAD - VOCAB), (0, VPAD - VOCAB)))
           .reshape(VPAD * VPAD))

    out1, parts = _sc_gather(tt8, table.reshape(VOCAB * VOCAB),
                             XT, Xf, Yf)
    lse_sum = _lse_sum(Xi.reshape(NBLK, BLK, 1), table)
    loss = (lse_sum[0, 0] - jnp.sum(parts)) / TOK

    logits = (out1.reshape(L, NVT, NBT, 8, 128)
              .transpose(2, 4, 0, 1, 3)
              .reshape(B, L, VOCAB))
    return logits, loss
